# baseline (speedup 1.0000x reference)
import jax
import jax.numpy as jnp
from jax import lax
from jax.experimental import pallas as pl
from jax.experimental.pallas import tpu as pltpu


def _body(
    k_ref, mem_ref, wb_ref, bb_ref, out_ref, act_sc, p_sc, bits_sc
):
    B, T, TOPO = p_sc.shape
    D = mem_ref.shape[2]
    pid = pl.program_id(0)

    @pl.when(pid <= B)
    def _mlp():
        mem = mem_ref[0]
        act_prev = jnp.where(pid > 0, act_sc[...], 0.0)
        x = jnp.concatenate([mem, act_prev], axis=1)
        y = (
            jnp.dot(x, wb_ref[...], preferred_element_type=jnp.float32)
            + bb_ref[...]
        )
        h = lax.slice(y, (0, 0), (T, D))
        act_sc[...] = jnp.where(h >= 0.0, h, 0.01 * h)
        p = jnp.tanh(lax.slice(y, (0, D), (T, D + TOPO)))

        ones_t = jnp.ones((1, T), jnp.float32)
        c = jnp.dot(ones_t, p, preferred_element_type=jnp.float32) * (1.0 / T)
        q2 = (p - c) * (p - c)
        ones_d = jnp.ones((1, TOPO), jnp.float32)
        d2 = lax.dot_general(
            ones_d, q2, (((1,), (1,)), ((), ())), preferred_element_type=jnp.float32
        )
        dist = jnp.sqrt(d2)
        idx = jnp.maximum(pid - 1, 0)
        p_sc[idx] = p.astype(jnp.bfloat16)
        bits_sc[idx] = lax.bitcast_convert_type(dist, jnp.int32)

    @pl.when(pid == B + 1)
    def _select():
        bits = jnp.concatenate([bits_sc[b] for b in range(B)], axis=0)
        kv = k_ref[...]

        def step(i, res):
            cand = res | (jnp.int32(1) << (jnp.int32(30) - i))
            cnt = jnp.sum((bits < cand).astype(jnp.int32), axis=1, keepdims=True)
            return jnp.where(cnt >= kv, res, cand)

        res = lax.fori_loop(0, 31, step, jnp.zeros((B, 1), jnp.int32))

        maskf = (bits <= res).astype(jnp.float32)
        counts = jnp.maximum(jnp.sum(maskf, axis=1, keepdims=True), 1.0)
        for b in range(B):
            mrow = lax.slice(maskf, (b, 0), (b + 1, T)).astype(jnp.bfloat16)
            cb = lax.slice(counts, (b, 0), (b + 1, 1))
            rb = jnp.dot(mrow, p_sc[b], preferred_element_type=jnp.float32)
            out_ref[pl.ds(b, 1), :] = rb / cb


def kernel(memories, importance, W1, b1, W2, b2):
    B, T, D = memories.shape
    TOPO = W2.shape[1]
    k = jnp.clip(jnp.round(importance * (T - 1) + 1.0).astype(jnp.int32), 1, T)
    wbig = jnp.zeros((D + TOPO, D + TOPO), jnp.float32)
    wbig = wbig.at[:D, :D].set(W1).at[D:, D:].set(W2)
    bbig = jnp.concatenate([b1, b2]).reshape(1, D + TOPO)
    return pl.pallas_call(
        _body,
        grid=(B + 2,),
        in_specs=[
            pl.BlockSpec((B, 1), lambda i: (0, 0)),
            pl.BlockSpec((1, T, D), lambda i: (jnp.minimum(i, B - 1), 0, 0)),
            pl.BlockSpec((D + TOPO, D + TOPO), lambda i: (0, 0)),
            pl.BlockSpec((1, D + TOPO), lambda i: (0, 0)),
        ],
        out_specs=pl.BlockSpec((B, TOPO), lambda i: (0, 0)),
        out_shape=jax.ShapeDtypeStruct((B, TOPO), jnp.float32),
        scratch_shapes=[
            pltpu.VMEM((T, TOPO), jnp.float32),
            pltpu.VMEM((B, T, TOPO), jnp.bfloat16),
            pltpu.VMEM((B, 1, T), jnp.int32),
        ],
        compiler_params=pltpu.CompilerParams(
            vmem_limit_bytes=100 * 1024 * 1024,
        ),
    )(k, memories, wbig, bbig)


# speedup vs baseline: 1.4153x; 1.4153x over previous
import jax
import jax.numpy as jnp
from jax import lax
from jax.experimental import pallas as pl
from jax.experimental.pallas import tpu as pltpu


def _body(k_ref, mem_ref, w1_ref, b1_ref, w2_ref, b2_ref, out_ref, p_sc, bits_sc):
    B, T, _ = p_sc.shape
    pid = pl.program_id(0)

    @pl.when(pid < B)
    def _mlp():
        mem = mem_ref[0]
        h = jnp.dot(mem, w1_ref[...], preferred_element_type=jnp.float32)
        h += b1_ref[...]
        h = jnp.where(h >= 0.0, h, 0.01 * h)
        p = jnp.tanh(
            jnp.dot(h, w2_ref[...], preferred_element_type=jnp.float32) + b2_ref[...]
        )

        ones_t = jnp.ones((1, T), jnp.float32)
        c = jnp.dot(ones_t, p, preferred_element_type=jnp.float32) * (1.0 / T)
        q2 = (p - c) * (p - c)
        ones_d = jnp.ones((1, q2.shape[1]), jnp.float32)
        d2 = lax.dot_general(
            ones_d, q2, (((1,), (1,)), ((), ())), preferred_element_type=jnp.float32
        )
        dist = jnp.sqrt(d2)
        p_sc[pid] = p.astype(jnp.bfloat16)
        bits_sc[pid] = lax.bitcast_convert_type(dist, jnp.int32)

    @pl.when(pid == B)
    def _select():
        bits = jnp.concatenate([bits_sc[b] for b in range(B)], axis=0)
        kv = k_ref[...]

        def step(i, res):
            cand = res | (jnp.int32(1) << (jnp.int32(30) - i))
            cnt = jnp.sum((bits < cand).astype(jnp.int32), axis=1, keepdims=True)
            return jnp.where(cnt >= kv, res, cand)

        res = lax.fori_loop(0, 31, step, jnp.zeros((B, 1), jnp.int32))

        maskf = (bits <= res).astype(jnp.float32)
        counts = jnp.maximum(jnp.sum(maskf, axis=1, keepdims=True), 1.0)
        for b in range(B):
            mrow = lax.slice(maskf, (b, 0), (b + 1, T)).astype(jnp.bfloat16)
            cb = lax.slice(counts, (b, 0), (b + 1, 1))
            rb = jnp.dot(mrow, p_sc[b], preferred_element_type=jnp.float32)
            out_ref[pl.ds(b, 1), :] = rb / cb


def kernel(memories, importance, W1, b1, W2, b2):
    B, T, D = memories.shape
    TOPO = W2.shape[1]
    k = jnp.clip(
        jnp.round(importance * (T - 1) + 1.0).astype(jnp.int32), 1, T
    )
    return pl.pallas_call(
        _body,
        grid=(B + 1,),
        in_specs=[
            pl.BlockSpec((B, 1), lambda i: (0, 0)),
            pl.BlockSpec((1, T, D), lambda i: (jnp.minimum(i, B - 1), 0, 0)),
            pl.BlockSpec((D, TOPO), lambda i: (0, 0)),
            pl.BlockSpec((1, TOPO), lambda i: (0, 0)),
            pl.BlockSpec((TOPO, TOPO), lambda i: (0, 0)),
            pl.BlockSpec((1, TOPO), lambda i: (0, 0)),
        ],
        out_specs=pl.BlockSpec((B, TOPO), lambda i: (0, 0)),
        out_shape=jax.ShapeDtypeStruct((B, TOPO), jnp.float32),
        scratch_shapes=[
            pltpu.VMEM((B, T, TOPO), jnp.bfloat16),
            pltpu.VMEM((B, 1, T), jnp.int32),
        ],
        compiler_params=pltpu.CompilerParams(
            vmem_limit_bytes=100 * 1024 * 1024,
        ),
    )(k, memories, W1, b1.reshape(1, TOPO), W2, b2.reshape(1, TOPO))


# speedup vs baseline: 1.5053x; 1.0636x over previous
import jax
import jax.numpy as jnp
from jax import lax
from jax.experimental import pallas as pl
from jax.experimental.pallas import tpu as pltpu


def _body(k_ref, mem_ref, w1_ref, b1_ref, w2_ref, b2_ref, out_ref, p_sc, bits_sc):
    B, T, _ = p_sc.shape
    NB = B // 2
    pid = pl.program_id(0)

    @pl.when(pid < NB)
    def _mlp():
        for j in range(2):
            mem = mem_ref[j]
            h = jnp.dot(mem, w1_ref[...], preferred_element_type=jnp.float32)
            h += b1_ref[...]
            h = jnp.maximum(h, 0.01 * h)
            p = jnp.tanh(
                jnp.dot(h, w2_ref[...], preferred_element_type=jnp.float32)
                + b2_ref[...]
            )

            ones_t = jnp.ones((1, T), jnp.float32)
            c = jnp.dot(ones_t, p, preferred_element_type=jnp.float32) * (1.0 / T)
            q2 = (p - c) * (p - c)
            ones_d = jnp.ones((1, q2.shape[1]), jnp.float32)
            d2 = lax.dot_general(
                ones_d,
                q2,
                (((1,), (1,)), ((), ())),
                preferred_element_type=jnp.float32,
            )
            dist = jnp.sqrt(d2)
            p_sc[2 * pid + j] = p.astype(jnp.bfloat16)
            bits_sc[2 * pid + j] = lax.bitcast_convert_type(dist, jnp.int32)

    @pl.when(pid == NB)
    def _select():
        bits = jnp.concatenate([bits_sc[b] for b in range(B)], axis=0)
        kv = k_ref[...]

        def step(i, res):
            cand = res | (jnp.int32(1) << (jnp.int32(30) - i))
            cnt = jnp.sum((bits < cand).astype(jnp.int32), axis=1, keepdims=True)
            return jnp.where(cnt >= kv, res, cand)

        res = lax.fori_loop(0, 31, step, jnp.zeros((B, 1), jnp.int32))

        maskf = (bits <= res).astype(jnp.float32)
        counts = jnp.maximum(jnp.sum(maskf, axis=1, keepdims=True), 1.0)
        for b in range(B):
            mrow = lax.slice(maskf, (b, 0), (b + 1, T)).astype(jnp.bfloat16)
            cb = lax.slice(counts, (b, 0), (b + 1, 1))
            rb = jnp.dot(mrow, p_sc[b], preferred_element_type=jnp.float32)
            out_ref[pl.ds(b, 1), :] = rb / cb


def kernel(memories, importance, W1, b1, W2, b2):
    B, T, D = memories.shape
    TOPO = W2.shape[1]
    k = jnp.clip(
        jnp.round(importance * (T - 1) + 1.0).astype(jnp.int32), 1, T
    )
    return pl.pallas_call(
        _body,
        grid=(B // 2 + 1,),
        in_specs=[
            pl.BlockSpec((B, 1), lambda i: (0, 0)),
            pl.BlockSpec((2, T, D), lambda i: (jnp.minimum(i, B // 2 - 1), 0, 0)),
            pl.BlockSpec((D, TOPO), lambda i: (0, 0)),
            pl.BlockSpec((1, TOPO), lambda i: (0, 0)),
            pl.BlockSpec((TOPO, TOPO), lambda i: (0, 0)),
            pl.BlockSpec((1, TOPO), lambda i: (0, 0)),
        ],
        out_specs=pl.BlockSpec((B, TOPO), lambda i: (0, 0)),
        out_shape=jax.ShapeDtypeStruct((B, TOPO), jnp.float32),
        scratch_shapes=[
            pltpu.VMEM((B, T, TOPO), jnp.bfloat16),
            pltpu.VMEM((B, 1, T), jnp.int32),
        ],
        compiler_params=pltpu.CompilerParams(
            vmem_limit_bytes=100 * 1024 * 1024,
        ),
    )(k, memories, W1, b1.reshape(1, TOPO), W2, b2.reshape(1, TOPO))
